# baseline (speedup 1.0000x reference)
import functools

import jax
import jax.numpy as jnp
from jax import lax
from jax.experimental import pallas as pl
from jax.experimental.pallas import tpu as pltpu
from jax.experimental.pallas import tpu_sc as plsc

B, C, H, W = 16, 96, 224, 224
HW = H * W
N = B * HW
NC, NS, L = 2, 16, 16
NW = NC * NS
CHUNK = N // NW
GCH = 128
NG = CHUNK // GCH
VPR = GCH // L

_mesh = plsc.VectorSubcoreMesh(
    core_axis_name="c", subcore_axis_name="s", num_cores=NC, num_subcores=NS
)


@functools.partial(
    pl.kernel,
    out_type=jax.ShapeDtypeStruct((NW, 2 * L), jnp.float32),
    mesh=_mesh,
    scratch_types=[
        pltpu.VMEM((CHUNK,), jnp.int32),
        pltpu.VMEM((NG, GCH), jnp.int32),
        pltpu.VMEM((CHUNK,), jnp.float32),
        pltpu.VMEM((NG, GCH), jnp.float32),
        pltpu.VMEM((2 * L,), jnp.float32),
        pltpu.SemaphoreType.DMA,
    ],
)
def _traloss_sc(yhat_hbm, label_hbm, mask_hbm, parts_hbm,
                lab_v, idx_v, mask_v, vals_v, part_v, sem):
    c = lax.axis_index("c")
    s = lax.axis_index("s")
    wid = c * NS + s
    base = wid * CHUNK
    b = wid // 2
    off = b * (C * HW) + (base - b * HW)
    lane = lax.iota(jnp.int32, L)

    pltpu.sync_copy(label_hbm.at[pl.ds(base, CHUNK)], lab_v)
    pltpu.sync_copy(mask_hbm.at[pl.ds(base, CHUNK)], mask_v)

    def idx_body(j, _):
        row = off + j * GCH
        for k in range(VPR):
            sl = pl.ds(k * L, L)
            idx_v[j, sl] = lab_v[pl.ds(j * GCH + k * L, L)] * HW + (row + k * L) + lane
        return 0

    lax.fori_loop(0, NG, idx_body, 0)

    def gather_body(j, _):
        pltpu.async_copy(yhat_hbm.at[idx_v.at[j]], vals_v.at[j], sem).wait()
        return 0

    lax.fori_loop(0, NG, gather_body, 0)

    def acc_body(j, carry):
        an, ad = carry
        for k in range(VPR):
            m = mask_v[pl.ds(j * GCH + k * L, L)]
            an = an + vals_v[j, pl.ds(k * L, L)] * m
            ad = ad + m
        return an, ad

    zero = jnp.zeros((L,), jnp.float32)
    an, ad = lax.fori_loop(0, NG, acc_body, (zero, zero))

    part_v[pl.ds(0, L)] = an
    part_v[pl.ds(L, L)] = ad
    pltpu.sync_copy(part_v, parts_hbm.at[wid])


def _fold_body(parts_ref, o_ref):
    x = parts_ref[...]
    rn = jnp.sum(x[:, :L], axis=1)
    rd = jnp.sum(x[:, L:], axis=1)
    row = lax.broadcasted_iota(jnp.int32, (B, NW), 1)
    samp = lax.broadcasted_iota(jnp.int32, (B, NW), 0)
    sel = jnp.where(row // 2 == samp, 1.0, 0.0)
    num = jnp.sum(sel * rn[None, :], axis=1)
    den = jnp.sum(sel * rd[None, :], axis=1)
    o_ref[0, 0] = jnp.sum(num / den) * (1.0 / B)


_fold = pl.pallas_call(
    _fold_body,
    out_shape=jax.ShapeDtypeStruct((1, 1), jnp.float32),
    out_specs=pl.BlockSpec(memory_space=pltpu.SMEM),
)


def kernel(y_hat, label, mask):
    yf = y_hat.reshape(-1)
    lf = label.astype(jnp.int32).reshape(-1)
    mf = mask.reshape(-1)
    parts = _traloss_sc(yf, lf, mf)
    return _fold(parts)[0, 0]


# speedup vs baseline: 1.2399x; 1.2399x over previous
import functools

import jax
import jax.numpy as jnp
from jax import lax
from jax.experimental import pallas as pl
from jax.experimental.pallas import tpu as pltpu
from jax.experimental.pallas import tpu_sc as plsc

B, C, H, W = 16, 96, 224, 224
HW = H * W
N = B * HW
NC, NS, L = 2, 16, 16
NW = NC * NS
CHUNK = N // NW
NV = CHUNK // L

_mesh = plsc.VectorSubcoreMesh(
    core_axis_name="c", subcore_axis_name="s", num_cores=NC, num_subcores=NS
)


@functools.partial(
    pl.kernel,
    out_type=jax.ShapeDtypeStruct((NW, 2 * L), jnp.float32),
    mesh=_mesh,
    scratch_types=[
        pltpu.VMEM((CHUNK,), jnp.int32),
        pltpu.VMEM((CHUNK,), jnp.int32),
        pltpu.VMEM((CHUNK,), jnp.float32),
        pltpu.VMEM((CHUNK,), jnp.float32),
        pltpu.VMEM((2 * L,), jnp.float32),
        pltpu.SemaphoreType.DMA,
    ],
)
def _traloss_sc(yhat_hbm, label_hbm, mask_hbm, parts_hbm,
                lab_v, idx_v, mask_v, vals_v, part_v, sem):
    c = lax.axis_index("c")
    s = lax.axis_index("s")
    wid = c * NS + s
    base = wid * CHUNK
    b = wid // 2
    off = b * (C * HW) + (base - b * HW)
    lane = lax.iota(jnp.int32, L)

    pltpu.sync_copy(label_hbm.at[pl.ds(base, CHUNK)], lab_v)
    pltpu.sync_copy(mask_hbm.at[pl.ds(base, CHUNK)], mask_v)

    def idx_body(i, _):
        sl = pl.ds(i * L, L)
        idx_v[sl] = lab_v[sl] * HW + (off + i * L) + lane
        return 0

    lax.fori_loop(0, NV, idx_body, 0, unroll=8)

    pltpu.async_copy(yhat_hbm.at[idx_v], vals_v, sem).wait()

    def acc_body(i, carry):
        an, ad = carry
        sl = pl.ds(i * L, L)
        m = mask_v[sl]
        return an + vals_v[sl] * m, ad + m

    zero = jnp.zeros((L,), jnp.float32)
    an, ad = lax.fori_loop(0, NV, acc_body, (zero, zero), unroll=8)

    part_v[pl.ds(0, L)] = an
    part_v[pl.ds(L, L)] = ad
    pltpu.sync_copy(part_v, parts_hbm.at[wid])


def _fold_body(parts_ref, o_ref):
    x = parts_ref[...]
    rn = jnp.sum(x[:, :L], axis=1)
    rd = jnp.sum(x[:, L:], axis=1)
    row = lax.broadcasted_iota(jnp.int32, (B, NW), 1)
    samp = lax.broadcasted_iota(jnp.int32, (B, NW), 0)
    sel = jnp.where(row // 2 == samp, 1.0, 0.0)
    num = jnp.sum(sel * rn[None, :], axis=1)
    den = jnp.sum(sel * rd[None, :], axis=1)
    o_ref[0, 0] = jnp.sum(num / den) * (1.0 / B)


_fold = pl.pallas_call(
    _fold_body,
    out_shape=jax.ShapeDtypeStruct((1, 1), jnp.float32),
    out_specs=pl.BlockSpec(memory_space=pltpu.SMEM),
)


def kernel(y_hat, label, mask):
    yf = y_hat.reshape(-1)
    lf = label.astype(jnp.int32).reshape(-1)
    mf = mask.reshape(-1)
    parts = _traloss_sc(yf, lf, mf)
    return _fold(parts)[0, 0]


# speedup vs baseline: 2.8738x; 2.3178x over previous
import jax
import jax.numpy as jnp
from jax import lax
from jax.experimental import pallas as pl
from jax.experimental.pallas import tpu as pltpu

B, C, H, W = 16, 96, 224, 224
HB = 16
NH = H // HB


def _body(y_ref, lab_ref, m_ref, o_ref):
    b = pl.program_id(0)
    h = pl.program_id(1)
    lab = lab_ref[0, 0]
    m = m_ref[0, 0]
    acc = jnp.zeros((HB, W), jnp.float32)
    for c in range(C):
        acc = acc + jnp.where(lab == c, y_ref[0, c], 0.0)
    s_num = jnp.sum(acc * m)
    s_den = jnp.sum(m)

    @pl.when(h == 0)
    def _():
        o_ref[b, 0] = s_num
        o_ref[b, 1] = s_den

    @pl.when(h != 0)
    def _():
        o_ref[b, 0] = o_ref[b, 0] + s_num
        o_ref[b, 1] = o_ref[b, 1] + s_den


_dense = pl.pallas_call(
    _body,
    grid=(B, NH),
    in_specs=[
        pl.BlockSpec((1, C, HB, W), lambda b, h: (b, 0, h, 0)),
        pl.BlockSpec((1, 1, HB, W), lambda b, h: (b, 0, h, 0)),
        pl.BlockSpec((1, 1, HB, W), lambda b, h: (b, 0, h, 0)),
    ],
    out_specs=pl.BlockSpec(memory_space=pltpu.SMEM),
    out_shape=jax.ShapeDtypeStruct((B, 2), jnp.float32),
)


def _fold_body(parts_ref, o_ref):
    x = parts_ref[...]
    o_ref[0, 0] = jnp.sum(x[:, 0:1] / x[:, 1:2]) * (1.0 / B)


_fold = pl.pallas_call(
    _fold_body,
    out_shape=jax.ShapeDtypeStruct((1, 1), jnp.float32),
    out_specs=pl.BlockSpec(memory_space=pltpu.SMEM),
)


def kernel(y_hat, label, mask):
    parts = _dense(y_hat, label.astype(jnp.int32), mask)
    return _fold(parts)[0, 0]


# speedup vs baseline: 3.2441x; 1.1289x over previous
import functools

import jax
import jax.numpy as jnp
from jax import lax
from jax.experimental import pallas as pl
from jax.experimental.pallas import tpu as pltpu
from jax.experimental.pallas import tpu_sc as plsc

B, C, H, W = 16, 96, 224, 224
NC, NS, L = 2, 16, 16
NW = NC * NS
CH = C // 2
UNITS = 14
VPR_W = W // L

_mesh = plsc.VectorSubcoreMesh(
    core_axis_name="c", subcore_axis_name="s", num_cores=NC, num_subcores=NS
)


@functools.partial(
    pl.kernel,
    out_type=jax.ShapeDtypeStruct((NW, 2 * L), jnp.float32),
    mesh=_mesh,
    compiler_params=pltpu.CompilerParams(
        use_tc_tiling_on_sc=True, needs_layout_passes=False
    ),
    scratch_types=[
        pltpu.VMEM((CH, 8, W), jnp.float32),
        pltpu.VMEM((8, W), jnp.int32),
        pltpu.VMEM((8, W), jnp.float32),
        pltpu.VMEM((2 * L,), jnp.float32),
    ],
)
def _traloss_sc(yhat_hbm, label_hbm, mask_hbm, parts_hbm,
                y_v, lab_v, m_v, part_v):
    c = lax.axis_index("c")
    s = lax.axis_index("s")
    wid = c * NS + s
    b = lax.shift_right_logical(wid, 1)
    hb0 = UNITS * (wid & 1)
    lane = lax.iota(jnp.int32, L)
    zero = jnp.zeros((L,), jnp.float32)

    def unit_body(u, carry):
        an, ad = carry
        h8 = (hb0 + u) * 8
        pltpu.sync_copy(label_hbm.at[b, 0, pl.ds(h8, 8)], lab_v)
        pltpu.sync_copy(mask_hbm.at[b, 0, pl.ds(h8, 8)], m_v)
        for half in range(2):
            c0 = CH * half
            pltpu.sync_copy(yhat_hbm.at[b, pl.ds(c0, CH), pl.ds(h8, 8)], y_v)
            for r in range(8):
                for k in range(VPR_W):
                    cs = k * L
                    lab = lab_v[r, pl.ds(cs, L)]
                    cc = lab - c0
                    inr = (cc >= 0) & (cc < CH)
                    cidx = jnp.clip(cc, 0, CH - 1)
                    hidx = jnp.full((L,), r, jnp.int32)
                    widx = cs + lane
                    g = plsc.load_gather(y_v, [cidx, hidx, widx])
                    m = m_v[r, pl.ds(cs, L)]
                    an = an + jnp.where(inr, g, 0.0) * m
                    if half == 0:
                        ad = ad + m
        return an, ad

    an, ad = lax.fori_loop(0, UNITS, unit_body, (zero, zero))

    part_v[pl.ds(0, L)] = an
    part_v[pl.ds(L, L)] = ad
    pltpu.sync_copy(part_v, parts_hbm.at[wid])


def _fold_body(parts_ref, o_ref):
    x = parts_ref[...]
    rn = jnp.sum(x[:, :L], axis=1)
    rd = jnp.sum(x[:, L:], axis=1)
    row = lax.broadcasted_iota(jnp.int32, (B, NW), 1)
    samp = lax.broadcasted_iota(jnp.int32, (B, NW), 0)
    sel = jnp.where(row // 2 == samp, 1.0, 0.0)
    num = jnp.sum(sel * rn[None, :], axis=1)
    den = jnp.sum(sel * rd[None, :], axis=1)
    o_ref[0, 0] = jnp.sum(num / den) * (1.0 / B)


_fold = pl.pallas_call(
    _fold_body,
    out_shape=jax.ShapeDtypeStruct((1, 1), jnp.float32),
    out_specs=pl.BlockSpec(memory_space=pltpu.SMEM),
)


def kernel(y_hat, label, mask):
    parts = _traloss_sc(y_hat, label.astype(jnp.int32), mask)
    return _fold(parts)[0, 0]


# speedup vs baseline: 3.7622x; 1.1597x over previous
import functools

import jax
import jax.numpy as jnp
from jax import lax
from jax.experimental import pallas as pl
from jax.experimental.pallas import tpu as pltpu
from jax.experimental.pallas import tpu_sc as plsc

B, C, H, W = 16, 96, 224, 224
NC, NS, L = 2, 16, 16
NW = NC * NS
CQ = C // 4
UNITS = 14
NBLK = UNITS * 4
VPR_W = W // L

_mesh = plsc.VectorSubcoreMesh(
    core_axis_name="c", subcore_axis_name="s", num_cores=NC, num_subcores=NS
)


@functools.partial(
    pl.kernel,
    out_type=jax.ShapeDtypeStruct((NW, 2 * L), jnp.float32),
    mesh=_mesh,
    compiler_params=pltpu.CompilerParams(
        use_tc_tiling_on_sc=True, needs_layout_passes=False
    ),
    scratch_types=[
        pltpu.VMEM((CQ, 8, W), jnp.float32),
        pltpu.VMEM((CQ, 8, W), jnp.float32),
        pltpu.VMEM((8, W), jnp.int32),
        pltpu.VMEM((8, W), jnp.float32),
        pltpu.VMEM((2 * L,), jnp.float32),
        pltpu.SemaphoreType.DMA,
        pltpu.SemaphoreType.DMA,
    ],
)
def _traloss_sc(yhat_hbm, label_hbm, mask_hbm, parts_hbm,
                y0_v, y1_v, lab_v, m_v, part_v, sem0, sem1):
    c = lax.axis_index("c")
    s = lax.axis_index("s")
    wid = c * NS + s
    b = lax.shift_right_logical(wid, 1)
    hb0 = UNITS * (wid & 1)
    lane = lax.iota(jnp.int32, L)
    zero = jnp.zeros((L,), jnp.float32)

    def src(g):
        h8 = (hb0 + lax.shift_right_logical(g, 2)) * 8
        c0 = CQ * (g & 3)
        return yhat_hbm.at[b, pl.ds(c0, CQ), pl.ds(h8, 8)]

    def fire(g, buf, sem):
        pltpu.async_copy(src(g), buf, sem)

    def drain(g, buf, sem):
        pltpu.make_async_copy(src(g), buf, sem).wait()

    def compute(g, buf, carry):
        an, ad = carry
        c0 = CQ * (g & 3)
        first_q = (g & 3) == 0
        for r in range(8):
            for k in range(VPR_W):
                cs = k * L
                lab = lab_v[r, pl.ds(cs, L)]
                cc = lab - c0
                inr = (cc >= 0) & (cc < CQ)
                cidx = jnp.clip(cc, 0, CQ - 1)
                hidx = jnp.full((L,), r, jnp.int32)
                widx = cs + lane
                g_val = plsc.load_gather(buf, [cidx, hidx, widx])
                m = m_v[r, pl.ds(cs, L)]
                an = an + jnp.where(inr, g_val, 0.0) * m
                ad = ad + jnp.where(first_q, m, 0.0)
        return an, ad

    def unit_prefetch_labels(g):
        h8 = (hb0 + lax.shift_right_logical(g, 2)) * 8
        pltpu.sync_copy(label_hbm.at[b, 0, pl.ds(h8, 8)], lab_v)
        pltpu.sync_copy(mask_hbm.at[b, 0, pl.ds(h8, 8)], m_v)

    fire(0, y0_v, sem0)

    def pair_body(j, carry):
        g0 = 2 * j

        @pl.when(g0 + 1 < NBLK)
        def _():
            fire(g0 + 1, y1_v, sem1)

        unit_prefetch_labels(g0)
        drain(g0, y0_v, sem0)
        carry = compute(g0, y0_v, carry)

        @pl.when(g0 + 2 < NBLK)
        def _():
            fire(g0 + 2, y0_v, sem0)

        unit_prefetch_labels(g0 + 1)
        drain(g0 + 1, y1_v, sem1)
        carry = compute(g0 + 1, y1_v, carry)
        return carry

    an, ad = lax.fori_loop(0, NBLK // 2, pair_body, (zero, zero))

    part_v[pl.ds(0, L)] = an
    part_v[pl.ds(L, L)] = ad
    pltpu.sync_copy(part_v, parts_hbm.at[wid])


def _fold_body(parts_ref, o_ref):
    x = parts_ref[...]
    rn = jnp.sum(x[:, :L], axis=1)
    rd = jnp.sum(x[:, L:], axis=1)
    row = lax.broadcasted_iota(jnp.int32, (B, NW), 1)
    samp = lax.broadcasted_iota(jnp.int32, (B, NW), 0)
    sel = jnp.where(row // 2 == samp, 1.0, 0.0)
    num = jnp.sum(sel * rn[None, :], axis=1)
    den = jnp.sum(sel * rd[None, :], axis=1)
    o_ref[0, 0] = jnp.sum(num / den) * (1.0 / B)


_fold = pl.pallas_call(
    _fold_body,
    out_shape=jax.ShapeDtypeStruct((1, 1), jnp.float32),
    out_specs=pl.BlockSpec(memory_space=pltpu.SMEM),
)


def kernel(y_hat, label, mask):
    parts = _traloss_sc(y_hat, label.astype(jnp.int32), mask)
    return _fold(parts)[0, 0]
